# baseline (speedup 1.0000x reference)
import functools

import jax
import jax.numpy as jnp
from jax import lax
from jax.experimental import pallas as pl
from jax.experimental.pallas import tpu as pltpu

DIM = 64
NUM_HEADS = 4
HEAD_DIM = DIM // NUM_HEADS
TOP_K = 16
CHUNK = 64
TEMP = 0.1
B = 1024
P = 1000000
PC = P // CHUNK
CBLK = 125
NBLK = PC // CBLK
RBLK = 64


def _compress_kernel(pv_ref, ev_ref, wkr_ref, comp_ref, khn_ref):
    pv = pv_ref[...].reshape(CBLK, CHUNK, DIM)
    ev = ev_ref[0].astype(jnp.float32) + 1e-8
    w = ev / jnp.sum(ev, axis=-1, keepdims=True)
    comp = jnp.sum(pv * w[..., None], axis=1)
    n = jnp.sqrt(jnp.sum(comp * comp, axis=-1, keepdims=True))
    comp = comp / (n + 1e-12)
    comp_ref[0] = comp
    kr = lax.dot_general(comp.astype(jnp.bfloat16),
                         wkr_ref[...].astype(jnp.bfloat16),
                         (((1,), (1,)), ((), ())),
                         preferred_element_type=jnp.float32)
    ri = lax.broadcasted_iota(jnp.int32, (DIM, DIM), 0) // HEAD_DIM
    ci = lax.broadcasted_iota(jnp.int32, (DIM, DIM), 1) // HEAD_DIM
    seg = (ri == ci).astype(jnp.float32)
    hsum = lax.dot_general(kr * kr, seg, (((1,), (0,)), ((), ())),
                           preferred_element_type=jnp.float32, precision=lax.Precision.HIGHEST)
    khn_ref[0] = (kr / (jnp.sqrt(hsum) + 1e-12)).astype(jnp.bfloat16)


def _score_kernel(h_ref, khn_ref, comp_ref, wqr_ref, wout_ref, o_ref,
                  s0_scr):
    qr = lax.dot_general(h_ref[...].astype(jnp.bfloat16),
                         wqr_ref[...].astype(jnp.bfloat16),
                         (((1,), (1,)), ((), ())),
                         preferred_element_type=jnp.float32)
    ri = lax.broadcasted_iota(jnp.int32, (DIM, DIM), 0) // HEAD_DIM
    ci = lax.broadcasted_iota(jnp.int32, (DIM, DIM), 1) // HEAD_DIM
    seg = (ri == ci).astype(jnp.float32)
    hsum = lax.dot_general(qr * qr, seg, (((1,), (0,)), ((), ())),
                           preferred_element_type=jnp.float32, precision=lax.Precision.HIGHEST)
    qhn = (qr / (jnp.sqrt(hsum) + 1e-12)).astype(jnp.bfloat16)
    scores = lax.dot_general(qhn, khn_ref[...], (((1,), (1,)), ((), ())),
                             preferred_element_type=jnp.float32)
    scores = (scores / NUM_HEADS) / TEMP
    s0_scr[...] = scores
    m = None
    m1 = None
    for k in range(TOP_K):
        sv = s0_scr[...]
        cand = sv if k == 0 else jnp.where(sv < m, sv, -jnp.inf)
        m = jnp.max(cand, axis=1, keepdims=True)
        if k == 0:
            m1 = m
    s0 = s0_scr[...]
    ind = s0 >= m
    e = jnp.where(ind, jnp.exp(s0 - m1), 0.0)
    z = jnp.sum(e, axis=1, keepdims=True)
    w1 = e / z
    retr = lax.dot_general(w1, comp_ref[...], (((1,), (0,)), ((), ())),
                           preferred_element_type=jnp.float32, precision=lax.Precision.HIGHEST)
    o_ref[...] = lax.dot_general(retr.astype(jnp.bfloat16),
                                 wout_ref[...].astype(jnp.bfloat16),
                                 (((1,), (1,)), ((), ())),
                                 preferred_element_type=jnp.float32)


@jax.jit
def kernel(h, prototypes, evidence, W_QR, W_KR, W_out):
    ev = evidence.reshape(NBLK, CBLK, CHUNK)

    comp3, khn3 = pl.pallas_call(
        _compress_kernel,
        grid=(NBLK,),
        in_specs=[
            pl.BlockSpec((CBLK * CHUNK, DIM), lambda i: (i, 0)),
            pl.BlockSpec((1, CBLK, CHUNK), lambda i: (i, 0, 0)),
            pl.BlockSpec((DIM, DIM), lambda i: (0, 0)),
        ],
        out_specs=[
            pl.BlockSpec((1, CBLK, DIM), lambda i: (i, 0, 0)),
            pl.BlockSpec((1, CBLK, DIM), lambda i: (i, 0, 0)),
        ],
        out_shape=[
            jax.ShapeDtypeStruct((NBLK, CBLK, DIM), jnp.float32),
            jax.ShapeDtypeStruct((NBLK, CBLK, DIM), jnp.bfloat16),
        ],
    )(prototypes, ev, W_KR)

    comp = comp3.reshape(PC, DIM)
    khn = khn3.reshape(PC, DIM)

    out = pl.pallas_call(
        _score_kernel,
        grid=(B // RBLK,),
        in_specs=[
            pl.BlockSpec((RBLK, DIM), lambda i: (i, 0)),
            pl.BlockSpec((PC, DIM), lambda i: (0, 0)),
            pl.BlockSpec((PC, DIM), lambda i: (0, 0)),
            pl.BlockSpec((DIM, DIM), lambda i: (0, 0)),
            pl.BlockSpec((DIM, DIM), lambda i: (0, 0)),
        ],
        out_specs=pl.BlockSpec((RBLK, DIM), lambda i: (i, 0)),
        out_shape=jax.ShapeDtypeStruct((B, DIM), jnp.float32),
        scratch_shapes=[
            pltpu.VMEM((RBLK, PC), jnp.float32),
        ],
    )(h, khn, comp, W_QR, W_out)
    return out


# speedup vs baseline: 1.1714x; 1.1714x over previous
import functools

import jax
import jax.numpy as jnp
from jax import lax
from jax.experimental import pallas as pl
from jax.experimental.pallas import tpu as pltpu

DIM = 64
NUM_HEADS = 4
HEAD_DIM = DIM // NUM_HEADS
TOP_K = 16
CHUNK = 64
TEMP = 0.1
B = 1024
P = 1000000
PC = P // CHUNK
CBLK = 125
NBLK = PC // CBLK
RBLK = 64


def _compress_kernel(pv_ref, ev_ref, wkr_ref, comp_ref, khn_ref):
    pv = pv_ref[...]
    ev = ev_ref[0].astype(jnp.float32) + 1e-8
    w = ev / jnp.sum(ev, axis=-1, keepdims=True)
    comp = jnp.sum(pv * w[..., None], axis=1)
    n = jnp.sqrt(jnp.sum(comp * comp, axis=-1, keepdims=True))
    comp = comp / (n + 1e-12)
    comp_ref[0] = comp
    kr = lax.dot_general(comp.astype(jnp.bfloat16),
                         wkr_ref[...].astype(jnp.bfloat16),
                         (((1,), (1,)), ((), ())),
                         preferred_element_type=jnp.float32)
    ri = lax.broadcasted_iota(jnp.int32, (DIM, DIM), 0) // HEAD_DIM
    ci = lax.broadcasted_iota(jnp.int32, (DIM, DIM), 1) // HEAD_DIM
    seg = (ri == ci).astype(jnp.float32)
    hsum = lax.dot_general(kr * kr, seg, (((1,), (0,)), ((), ())),
                           preferred_element_type=jnp.float32, precision=lax.Precision.HIGHEST)
    khn_ref[0] = (kr / (jnp.sqrt(hsum) + 1e-12)).astype(jnp.bfloat16)


def _score_kernel(h_ref, khn_ref, comp_ref, wqr_ref, wout_ref, o_ref,
                  s0_scr):
    qr = lax.dot_general(h_ref[...].astype(jnp.bfloat16),
                         wqr_ref[...].astype(jnp.bfloat16),
                         (((1,), (1,)), ((), ())),
                         preferred_element_type=jnp.float32)
    ri = lax.broadcasted_iota(jnp.int32, (DIM, DIM), 0) // HEAD_DIM
    ci = lax.broadcasted_iota(jnp.int32, (DIM, DIM), 1) // HEAD_DIM
    seg = (ri == ci).astype(jnp.float32)
    hsum = lax.dot_general(qr * qr, seg, (((1,), (0,)), ((), ())),
                           preferred_element_type=jnp.float32, precision=lax.Precision.HIGHEST)
    qhn = (qr / (jnp.sqrt(hsum) + 1e-12)).astype(jnp.bfloat16)
    scores = lax.dot_general(qhn, khn_ref[...], (((1,), (1,)), ((), ())),
                             preferred_element_type=jnp.float32)
    scores = (scores / NUM_HEADS) / TEMP
    s0_scr[...] = scores
    m = None
    m1 = None
    for k in range(TOP_K):
        sv = s0_scr[...]
        cand = sv if k == 0 else jnp.where(sv < m, sv, -jnp.inf)
        m = jnp.max(cand, axis=1, keepdims=True)
        if k == 0:
            m1 = m
    s0 = s0_scr[...]
    ind = s0 >= m
    e = jnp.where(ind, jnp.exp(s0 - m1), 0.0)
    z = jnp.sum(e, axis=1, keepdims=True)
    w1 = e / z
    retr = lax.dot_general(w1, comp_ref[...], (((1,), (0,)), ((), ())),
                           preferred_element_type=jnp.float32, precision=lax.Precision.HIGHEST)
    o_ref[...] = lax.dot_general(retr.astype(jnp.bfloat16),
                                 wout_ref[...].astype(jnp.bfloat16),
                                 (((1,), (1,)), ((), ())),
                                 preferred_element_type=jnp.float32)


@jax.jit
def kernel(h, prototypes, evidence, W_QR, W_KR, W_out):
    ev = evidence.reshape(NBLK, CBLK, CHUNK)
    pv3 = prototypes.reshape(PC, CHUNK, DIM)

    comp3, khn3 = pl.pallas_call(
        _compress_kernel,
        grid=(NBLK,),
        in_specs=[
            pl.BlockSpec((CBLK, CHUNK, DIM), lambda i: (i, 0, 0)),
            pl.BlockSpec((1, CBLK, CHUNK), lambda i: (i, 0, 0)),
            pl.BlockSpec((DIM, DIM), lambda i: (0, 0)),
        ],
        out_specs=[
            pl.BlockSpec((1, CBLK, DIM), lambda i: (i, 0, 0)),
            pl.BlockSpec((1, CBLK, DIM), lambda i: (i, 0, 0)),
        ],
        out_shape=[
            jax.ShapeDtypeStruct((NBLK, CBLK, DIM), jnp.float32),
            jax.ShapeDtypeStruct((NBLK, CBLK, DIM), jnp.bfloat16),
        ],
    )(pv3, ev, W_KR)

    comp = comp3.reshape(PC, DIM)
    khn = khn3.reshape(PC, DIM)

    out = pl.pallas_call(
        _score_kernel,
        grid=(B // RBLK,),
        in_specs=[
            pl.BlockSpec((RBLK, DIM), lambda i: (i, 0)),
            pl.BlockSpec((PC, DIM), lambda i: (0, 0)),
            pl.BlockSpec((PC, DIM), lambda i: (0, 0)),
            pl.BlockSpec((DIM, DIM), lambda i: (0, 0)),
            pl.BlockSpec((DIM, DIM), lambda i: (0, 0)),
        ],
        out_specs=pl.BlockSpec((RBLK, DIM), lambda i: (i, 0)),
        out_shape=jax.ShapeDtypeStruct((B, DIM), jnp.float32),
        scratch_shapes=[
            pltpu.VMEM((RBLK, PC), jnp.float32),
        ],
    )(h, khn, comp, W_QR, W_out)
    return out
